# baseline (speedup 1.0000x reference)
import functools

import jax
import jax.numpy as jnp
from jax import lax
from jax.experimental import pallas as pl
from jax.experimental.pallas import tpu as pltpu
from jax.experimental.pallas import tpu_sc as plsc

_B = 128
_P = 1024
_R = 256

_KS0 = 0
_KS1 = 42
_KS2 = _KS0 ^ _KS1 ^ 0x1BD11BDA
_ROT = ((13, 15, 26, 6), (17, 29, 16, 24))

_NW = 32
_ROWS_PER_W = _R // _NW
_L = 16
_NCHUNK = _P // _L
_SENTINEL = 0x3FFFFFFF


def _rotl(x, r):
    return lax.shift_left(x, jnp.int32(r)) | lax.shift_right_logical(
        x, jnp.int32(32 - r)
    )


def _threefry_bits(x1):
    ks = (jnp.int32(_KS0), jnp.int32(_KS1), jnp.int32(_KS2))
    x0 = jnp.full(x1.shape, ks[0], jnp.int32)
    x1 = x1 + ks[1]
    for g in range(5):
        for r in _ROT[g % 2]:
            x0 = x0 + x1
            x1 = _rotl(x1, r)
            x1 = x1 ^ x0
        x0 = x0 + ks[(g + 1) % 3]
        x1 = x1 + ks[(g + 2) % 3] + jnp.int32(g + 1)
    return x0 ^ x1


def _vals_kernel(o_ref):
    i = (
        lax.broadcasted_iota(jnp.int32, (_R, _P), 0) * _P
        + lax.broadcasted_iota(jnp.int32, (_R, _P), 1)
    )
    o_ref[:] = lax.shift_right_logical(_threefry_bits(i), 9)


def _splat_to_scalar(x):
    return jnp.max(x)


def _select_body(v_hbm, out_hbm, v_rows, mask_rows, hist, hist64, candbuf):
    wid = lax.axis_index("s") * 2 + lax.axis_index("c")
    base = pl.multiple_of(wid * _ROWS_PER_W, _ROWS_PER_W)
    lane = lax.iota(jnp.int32, _L)
    zeros = jnp.zeros((_L,), jnp.int32)
    ones = jnp.ones((_L,), jnp.int32)

    pltpu.sync_copy(v_hbm.at[pl.ds(base, _ROWS_PER_W)], v_rows)

    def zero_body(c, _):
        hist[pl.ds(pl.multiple_of(c * _L, _L), _L)] = zeros
        return 0

    lax.fori_loop(0, _NCHUNK, zero_body, 0, unroll=8)
    for q in range(4):
        hist64[pl.ds(q * _L, _L)] = zeros

    for j in range(_ROWS_PER_W):
        r = base + j
        k_row = jnp.where((r < _R // 2) == (j % 2 == 0), 768, 256)

        @plsc.parallel_loop(0, _NCHUNK, unroll=8)
        def _(c):
            vv = v_rows[j, pl.ds(pl.multiple_of(c * _L, _L), _L)]
            hi = lax.shift_right_logical(vv, 13)
            plsc.addupdate_scatter(hist, [hi], ones)
            plsc.addupdate_scatter(
                hist64, [lax.shift_right_logical(vv, 17)], ones
            )

        tot = jnp.int32(0)
        ch_vec = zeros
        lmax = zeros
        for q in range(4):
            s = plsc.cumsum(hist64[pl.ds(q * _L, _L)]) + tot
            ch_vec = ch_vec + plsc.all_reduce_population_count(s < k_row)
            lmax = jnp.maximum(lmax, jnp.where(s < k_row, s, 0))
            if q < 3:
                tot = jnp.max(s)
            hist64[pl.ds(q * _L, _L)] = zeros
        ch = jnp.max(ch_vec)
        l1 = jnp.max(lmax)

        s_fine = (
            plsc.cumsum(hist[pl.ds(pl.multiple_of(ch * _L, _L), _L)]) + l1
        )
        in_ch = s_fine < k_row
        g_local = plsc.all_reduce_population_count(in_ch)
        g = ch * _L + g_local
        nlt = jnp.max(jnp.where(lane < g_local, s_fine, l1))

        candbuf[:] = jnp.full((_L,), _SENTINEL, jnp.int32)

        @plsc.parallel_loop(0, _NCHUNK, unroll=8, carry=(zeros, lane))
        def _(c, carry):
            neq, pbase = carry
            off = pl.multiple_of(c * _L, _L)
            vv = v_rows[j, pl.ds(off, _L)]
            hist[pl.ds(off, _L)] = zeros
            hi = lax.shift_right_logical(vv, 13)
            m_lt = hi < g
            mask_rows[j, pl.ds(off, _L)] = m_lt.astype(jnp.int32)
            m_eq = hi == g
            meq_i = m_eq.astype(jnp.int32)
            excl = plsc.cumsum(meq_i) - meq_i
            idx = jnp.minimum(excl + neq, _L - 1)
            comb = (vv & 0x1FFF) * _P + pbase
            plsc.store_scatter(candbuf, [idx], comb, mask=m_eq)
            neq = neq + plsc.all_reduce_population_count(m_eq)
            return neq, pbase + _L

        sk, _ = plsc.sort_key_val(candbuf[:], candbuf[:])
        selm = lane < (k_row - nlt)
        plsc.store_scatter(
            mask_rows,
            [jnp.full((_L,), j, jnp.int32), sk & (_P - 1)],
            ones,
            mask=selm,
        )

    pltpu.sync_copy(mask_rows, out_hbm.at[pl.ds(base, _ROWS_PER_W)])


def _sc_select(v):
    mesh = plsc.VectorSubcoreMesh(core_axis_name="c", subcore_axis_name="s")
    f = pl.kernel(
        _select_body,
        out_type=jax.ShapeDtypeStruct((_R, _P), jnp.int32),
        mesh=mesh,
        compiler_params=pltpu.CompilerParams(
            needs_layout_passes=False, skip_device_barrier=True
        ),
        scratch_types=[
            pltpu.VMEM((_ROWS_PER_W, _P), jnp.int32),
            pltpu.VMEM((_ROWS_PER_W, _P), jnp.int32),
            pltpu.VMEM((_P,), jnp.int32),
            pltpu.VMEM((_NCHUNK,), jnp.int32),
            pltpu.VMEM((_L,), jnp.int32),
        ],
    )
    return f(v)


def kernel(x):
    del x
    v = pl.pallas_call(
        _vals_kernel,
        out_shape=jax.ShapeDtypeStruct((_R, _P), jnp.int32),
    )()
    mask = _sc_select(v)
    return mask.astype(jnp.bool_).reshape(_B, 2 * _P)


# speedup vs baseline: 1.0010x; 1.0010x over previous
import functools

import jax
import jax.numpy as jnp
from jax import lax
from jax.experimental import pallas as pl
from jax.experimental.pallas import tpu as pltpu
from jax.experimental.pallas import tpu_sc as plsc

_B = 128
_P = 1024
_R = 256

_KS0 = 0
_KS1 = 42
_KS2 = _KS0 ^ _KS1 ^ 0x1BD11BDA
_ROT = ((13, 15, 26, 6), (17, 29, 16, 24))

_NW = 32
_ROWS_PER_W = _R // _NW
_L = 16
_NCHUNK = _P // _L
_SENTINEL = 0x3FFFFFFF


def _rotl(x, r):
    return lax.shift_left(x, jnp.int32(r)) | lax.shift_right_logical(
        x, jnp.int32(32 - r)
    )


def _threefry_bits(x1):
    ks = (jnp.int32(_KS0), jnp.int32(_KS1), jnp.int32(_KS2))
    x0 = jnp.full(x1.shape, ks[0], jnp.int32)
    x1 = x1 + ks[1]
    for g in range(5):
        for r in _ROT[g % 2]:
            x0 = x0 + x1
            x1 = _rotl(x1, r)
            x1 = x1 ^ x0
        x0 = x0 + ks[(g + 1) % 3]
        x1 = x1 + ks[(g + 2) % 3] + jnp.int32(g + 1)
    return x0 ^ x1


def _vals_kernel(o_ref):
    i = (
        lax.broadcasted_iota(jnp.int32, (_R, _P), 0) * _P
        + lax.broadcasted_iota(jnp.int32, (_R, _P), 1)
    )
    o_ref[:] = lax.shift_right_logical(_threefry_bits(i), 9)


def _splat_to_scalar(x):
    return jnp.max(x)


def _select_body(v_hbm, out_hbm, v_rows, mask_rows, hist, hist64, candbuf):
    wid = lax.axis_index("s") * 2 + lax.axis_index("c")
    base = pl.multiple_of(wid * _ROWS_PER_W, _ROWS_PER_W)
    lane = lax.iota(jnp.int32, _L)
    zeros = jnp.zeros((_L,), jnp.int32)
    ones = jnp.ones((_L,), jnp.int32)

    pltpu.sync_copy(v_hbm.at[pl.ds(base, _ROWS_PER_W)], v_rows)

    def zero_body(c, _):
        hist[pl.ds(pl.multiple_of(c * _L, _L), _L)] = zeros
        return 0

    lax.fori_loop(0, _NCHUNK, zero_body, 0, unroll=8)
    for q in range(4):
        hist64[pl.ds(q * _L, _L)] = zeros

    for j in range(_ROWS_PER_W):
        r = base + j
        k_row = jnp.where((r < _R // 2) == (j % 2 == 0), 768, 256)

        @plsc.parallel_loop(0, _NCHUNK, unroll=8)
        def _(c):
            vv = v_rows[j, pl.ds(pl.multiple_of(c * _L, _L), _L)]
            hi = lax.shift_right_logical(vv, 13)
            plsc.addupdate_scatter(hist, [hi], ones)
            plsc.addupdate_scatter(
                hist64, [lax.shift_right_logical(vv, 17)], ones
            )

        tot = jnp.int32(0)
        ch_vec = zeros
        lmax = zeros
        for q in range(4):
            s = plsc.cumsum(hist64[pl.ds(q * _L, _L)]) + tot
            ch_vec = ch_vec + plsc.all_reduce_population_count(s < k_row)
            lmax = jnp.maximum(lmax, jnp.where(s < k_row, s, 0))
            if q < 3:
                tot = jnp.max(s)
            hist64[pl.ds(q * _L, _L)] = zeros
        ch = jnp.max(ch_vec)
        l1 = jnp.max(lmax)

        s_fine = (
            plsc.cumsum(hist[pl.ds(pl.multiple_of(ch * _L, _L), _L)]) + l1
        )
        in_ch = s_fine < k_row
        g_local = plsc.all_reduce_population_count(in_ch)
        g = ch * _L + g_local
        nlt = jnp.max(jnp.where(lane < g_local, s_fine, l1))

        candbuf[:] = jnp.full((_L,), _SENTINEL, jnp.int32)

        @plsc.parallel_loop(0, _NCHUNK, unroll=8, carry=(zeros, lane))
        def _(c, carry):
            neq, pbase = carry
            off = pl.multiple_of(c * _L, _L)
            vv = v_rows[j, pl.ds(off, _L)]
            hist[pl.ds(off, _L)] = zeros
            hi = lax.shift_right_logical(vv, 13)
            m_lt = hi < g
            mask_rows[j, pl.ds(off, _L)] = m_lt.astype(jnp.int32)
            m_eq = hi == g
            meq_i = m_eq.astype(jnp.int32)
            excl = plsc.cumsum(meq_i) - meq_i
            idx = jnp.minimum(excl + neq, _L - 1)
            comb = (vv & 0x1FFF) * _P + pbase
            plsc.store_scatter(candbuf, [idx], comb, mask=m_eq)
            neq = neq + plsc.all_reduce_population_count(m_eq)
            return neq, pbase + _L

        sk, _ = plsc.sort_key_val(candbuf[:], candbuf[:])
        selm = lane < (k_row - nlt)
        plsc.store_scatter(
            mask_rows,
            [jnp.full((_L,), j, jnp.int32), sk & (_P - 1)],
            ones,
            mask=selm,
        )

    pltpu.sync_copy(mask_rows, out_hbm.at[pl.ds(base, _ROWS_PER_W)])


def _sc_select(v):
    mesh = plsc.VectorSubcoreMesh(core_axis_name="c", subcore_axis_name="s")
    f = pl.kernel(
        _select_body,
        out_type=jax.ShapeDtypeStruct((_R, _P), jnp.int32),
        mesh=mesh,
        compiler_params=pltpu.CompilerParams(needs_layout_passes=False),
        scratch_types=[
            pltpu.VMEM((_ROWS_PER_W, _P), jnp.int32),
            pltpu.VMEM((_ROWS_PER_W, _P), jnp.int32),
            pltpu.VMEM((_P,), jnp.int32),
            pltpu.VMEM((_NCHUNK,), jnp.int32),
            pltpu.VMEM((_L,), jnp.int32),
        ],
    )
    return f(v)


def kernel(x):
    del x
    v = pl.pallas_call(
        _vals_kernel,
        out_shape=jax.ShapeDtypeStruct((_R, _P), jnp.int32),
    )()
    mask = _sc_select(v)
    return mask.astype(jnp.bool_).reshape(_B, 2 * _P)


# speedup vs baseline: 1.0600x; 1.0590x over previous
import functools

import jax
import jax.numpy as jnp
from jax import lax
from jax.experimental import pallas as pl
from jax.experimental.pallas import tpu as pltpu
from jax.experimental.pallas import tpu_sc as plsc

_B = 128
_P = 1024
_R = 256

_KS0 = 0
_KS1 = 42
_KS2 = _KS0 ^ _KS1 ^ 0x1BD11BDA
_ROT = ((13, 15, 26, 6), (17, 29, 16, 24))

_NW = 32
_ROWS_PER_W = _R // _NW
_L = 16
_NCHUNK = _P // _L
_SENTINEL = 0x3FFFFFFF


def _rotl(x, r):
    return lax.shift_left(x, jnp.int32(r)) | lax.shift_right_logical(
        x, jnp.int32(32 - r)
    )


def _threefry_bits(x1):
    ks = (jnp.int32(_KS0), jnp.int32(_KS1), jnp.int32(_KS2))
    x0 = jnp.full(x1.shape, ks[0], jnp.int32)
    x1 = x1 + ks[1]
    for g in range(5):
        for r in _ROT[g % 2]:
            x0 = x0 + x1
            x1 = _rotl(x1, r)
            x1 = x1 ^ x0
        x0 = x0 + ks[(g + 1) % 3]
        x1 = x1 + ks[(g + 2) % 3] + jnp.int32(g + 1)
    return x0 ^ x1


def _vals_kernel(o_ref):
    i = (
        lax.broadcasted_iota(jnp.int32, (_R, _P), 0) * _P
        + lax.broadcasted_iota(jnp.int32, (_R, _P), 1)
    )
    o_ref[:] = lax.shift_right_logical(_threefry_bits(i), 9)


def _splat_to_scalar(x):
    return jnp.max(x)


def _select_body(v_hbm, out_hbm, v_rows, mask_rows, hist, hist64, candbuf):
    wid = lax.axis_index("s") * 2 + lax.axis_index("c")
    base = pl.multiple_of(wid * _ROWS_PER_W, _ROWS_PER_W)
    lane = lax.iota(jnp.int32, _L)
    zeros = jnp.zeros((_L,), jnp.int32)
    ones = jnp.ones((_L,), jnp.int32)

    pltpu.sync_copy(v_hbm.at[pl.ds(base, _ROWS_PER_W)], v_rows)

    def zero_body(c, _):
        hist[pl.ds(pl.multiple_of(c * _L, _L), _L)] = zeros
        return 0

    lax.fori_loop(0, _NCHUNK, zero_body, 0, unroll=2)
    for q in range(4):
        hist64[pl.ds(q * _L, _L)] = zeros

    def row_body(j, _):
        r = base + j
        k_row = jnp.where((r < _R // 2) == (j % 2 == 0), 768, 256)

        @plsc.parallel_loop(0, _NCHUNK, unroll=8)
        def _(c):
            vv = v_rows[j, pl.ds(pl.multiple_of(c * _L, _L), _L)]
            hi = lax.shift_right_logical(vv, 13)
            plsc.addupdate_scatter(hist, [hi], ones)
            plsc.addupdate_scatter(
                hist64, [lax.shift_right_logical(vv, 17)], ones
            )

        tot = jnp.int32(0)
        ch_vec = zeros
        lmax = zeros
        for q in range(4):
            s = plsc.cumsum(hist64[pl.ds(q * _L, _L)]) + tot
            ch_vec = ch_vec + plsc.all_reduce_population_count(s < k_row)
            lmax = jnp.maximum(lmax, jnp.where(s < k_row, s, 0))
            if q < 3:
                tot = jnp.max(s)
            hist64[pl.ds(q * _L, _L)] = zeros
        ch = jnp.max(ch_vec)
        l1 = jnp.max(lmax)

        s_fine = (
            plsc.cumsum(hist[pl.ds(pl.multiple_of(ch * _L, _L), _L)]) + l1
        )
        in_ch = s_fine < k_row
        g_local = plsc.all_reduce_population_count(in_ch)
        g = ch * _L + g_local
        nlt = jnp.max(jnp.where(lane < g_local, s_fine, l1))

        candbuf[:] = jnp.full((_L,), _SENTINEL, jnp.int32)

        @plsc.parallel_loop(0, _NCHUNK, unroll=8, carry=(zeros, lane))
        def _(c, carry):
            neq, pbase = carry
            off = pl.multiple_of(c * _L, _L)
            vv = v_rows[j, pl.ds(off, _L)]
            hist[pl.ds(off, _L)] = zeros
            hi = lax.shift_right_logical(vv, 13)
            m_lt = hi < g
            mask_rows[j, pl.ds(off, _L)] = m_lt.astype(jnp.int32)
            m_eq = hi == g
            meq_i = m_eq.astype(jnp.int32)
            excl = plsc.cumsum(meq_i) - meq_i
            idx = jnp.minimum(excl + neq, _L - 1)
            comb = (vv & 0x1FFF) * _P + pbase
            plsc.store_scatter(candbuf, [idx], comb, mask=m_eq)
            neq = neq + plsc.all_reduce_population_count(m_eq)
            return neq, pbase + _L

        sk, _ = plsc.sort_key_val(candbuf[:], candbuf[:])
        selm = lane < (k_row - nlt)
        plsc.store_scatter(
            mask_rows,
            [jnp.full((_L,), j, jnp.int32), sk & (_P - 1)],
            ones,
            mask=selm,
        )
        return 0

    lax.fori_loop(0, _ROWS_PER_W, row_body, 0)

    pltpu.sync_copy(mask_rows, out_hbm.at[pl.ds(base, _ROWS_PER_W)])


def _sc_select(v):
    mesh = plsc.VectorSubcoreMesh(core_axis_name="c", subcore_axis_name="s")
    f = pl.kernel(
        _select_body,
        out_type=jax.ShapeDtypeStruct((_R, _P), jnp.int32),
        mesh=mesh,
        compiler_params=pltpu.CompilerParams(needs_layout_passes=False),
        scratch_types=[
            pltpu.VMEM((_ROWS_PER_W, _P), jnp.int32),
            pltpu.VMEM((_ROWS_PER_W, _P), jnp.int32),
            pltpu.VMEM((_P,), jnp.int32),
            pltpu.VMEM((_NCHUNK,), jnp.int32),
            pltpu.VMEM((_L,), jnp.int32),
        ],
    )
    return f(v)


def kernel(x):
    del x
    v = pl.pallas_call(
        _vals_kernel,
        out_shape=jax.ShapeDtypeStruct((_R, _P), jnp.int32),
    )()
    mask = _sc_select(v)
    return mask.astype(jnp.bool_).reshape(_B, 2 * _P)


# speedup vs baseline: 1.0768x; 1.0158x over previous
import functools

import jax
import jax.numpy as jnp
from jax import lax
from jax.experimental import pallas as pl
from jax.experimental.pallas import tpu as pltpu
from jax.experimental.pallas import tpu_sc as plsc

_B = 128
_P = 1024
_R = 256

_KS0 = 0
_KS1 = 42
_KS2 = _KS0 ^ _KS1 ^ 0x1BD11BDA
_ROT = ((13, 15, 26, 6), (17, 29, 16, 24))

_NW = 32
_ROWS_PER_W = _R // _NW
_L = 16
_NCHUNK = _P // _L
_SENTINEL = 0x3FFFFFFF


def _rotl(x, r):
    return lax.shift_left(x, jnp.int32(r)) | lax.shift_right_logical(
        x, jnp.int32(32 - r)
    )


def _threefry_bits(x1):
    ks = (jnp.int32(_KS0), jnp.int32(_KS1), jnp.int32(_KS2))
    x0 = jnp.full(x1.shape, ks[0], jnp.int32)
    x1 = x1 + ks[1]
    for g in range(5):
        for r in _ROT[g % 2]:
            x0 = x0 + x1
            x1 = _rotl(x1, r)
            x1 = x1 ^ x0
        x0 = x0 + ks[(g + 1) % 3]
        x1 = x1 + ks[(g + 2) % 3] + jnp.int32(g + 1)
    return x0 ^ x1


def _vals_kernel(o_ref):
    i = (
        lax.broadcasted_iota(jnp.int32, (_R, _P), 0) * _P
        + lax.broadcasted_iota(jnp.int32, (_R, _P), 1)
    )
    o_ref[:] = lax.shift_right_logical(_threefry_bits(i), 9)


def _splat_to_scalar(x):
    return jnp.max(x)


def _select_body(v_hbm, out_hbm, v_rows, mask_rows, hist, hist64, candbuf):
    wid = lax.axis_index("s") * 2 + lax.axis_index("c")
    base = pl.multiple_of(wid * _ROWS_PER_W, _ROWS_PER_W)
    lane = lax.iota(jnp.int32, _L)
    zeros = jnp.zeros((_L,), jnp.int32)
    ones = jnp.ones((_L,), jnp.int32)

    pltpu.sync_copy(v_hbm.at[pl.ds(base, _ROWS_PER_W)], v_rows)

    def zero_body(c, _):
        hist[pl.ds(pl.multiple_of(c * _L, _L), _L)] = zeros
        return 0

    lax.fori_loop(0, _NCHUNK, zero_body, 0, unroll=2)
    for q in range(4):
        hist64[pl.ds(q * _L, _L)] = zeros

    def row_body(j, _):
        r = base + j
        k_row = jnp.where((r < _R // 2) == (j % 2 == 0), 768, 256)
        jh = lax.shift_right_logical(j, 1)
        half = (j & 1) * _P

        @plsc.parallel_loop(0, _NCHUNK, unroll=8)
        def _(c):
            vv = v_rows[j, pl.ds(pl.multiple_of(c * _L, _L), _L)]
            hi = lax.shift_right_logical(vv, 13)
            plsc.addupdate_scatter(hist, [hi], ones)
            plsc.addupdate_scatter(
                hist64, [lax.shift_right_logical(vv, 17)], ones
            )

        tot = jnp.int32(0)
        ch_vec = zeros
        lmax = zeros
        for q in range(4):
            s = plsc.cumsum(hist64[pl.ds(q * _L, _L)]) + tot
            ch_vec = ch_vec + plsc.all_reduce_population_count(s < k_row)
            lmax = jnp.maximum(lmax, jnp.where(s < k_row, s, 0))
            if q < 3:
                tot = jnp.max(s)
            hist64[pl.ds(q * _L, _L)] = zeros
        ch = jnp.max(ch_vec)
        l1 = jnp.max(lmax)

        s_fine = (
            plsc.cumsum(hist[pl.ds(pl.multiple_of(ch * _L, _L), _L)]) + l1
        )
        in_ch = s_fine < k_row
        g_local = plsc.all_reduce_population_count(in_ch)
        g = ch * _L + g_local
        nlt = jnp.max(jnp.where(lane < g_local, s_fine, l1))

        candbuf[:] = jnp.full((_L,), _SENTINEL, jnp.int32)

        @plsc.parallel_loop(0, _NCHUNK, unroll=8, carry=(zeros, lane))
        def _(c, carry):
            neq, pbase = carry
            off = pl.multiple_of(c * _L, _L)
            vv = v_rows[j, pl.ds(off, _L)]
            hist[pl.ds(off, _L)] = zeros
            hi = lax.shift_right_logical(vv, 13)
            m_lt = hi < g
            mask_rows[jh, pl.ds(half + off, _L)] = m_lt.astype(jnp.int32)
            m_eq = hi == g
            meq_i = m_eq.astype(jnp.int32)
            excl = plsc.cumsum(meq_i) - meq_i
            idx = jnp.minimum(excl + neq, _L - 1)
            comb = (vv & 0x1FFF) * _P + pbase
            plsc.store_scatter(candbuf, [idx], comb, mask=m_eq)
            neq = neq + plsc.all_reduce_population_count(m_eq)
            return neq, pbase + _L

        sk, _ = plsc.sort_key_val(candbuf[:], candbuf[:])
        selm = lane < (k_row - nlt)
        plsc.store_scatter(
            mask_rows,
            [jnp.full((_L,), 0, jnp.int32) + jh, half + (sk & (_P - 1))],
            ones,
            mask=selm,
        )
        return 0

    lax.fori_loop(0, _ROWS_PER_W, row_body, 0)

    pltpu.sync_copy(
        mask_rows,
        out_hbm.at[pl.ds(pl.multiple_of(wid * (_ROWS_PER_W // 2), _ROWS_PER_W // 2), _ROWS_PER_W // 2)],
    )


def _sc_select(v):
    mesh = plsc.VectorSubcoreMesh(core_axis_name="c", subcore_axis_name="s")
    f = pl.kernel(
        _select_body,
        out_type=jax.ShapeDtypeStruct((_B, 2 * _P), jnp.int32),
        mesh=mesh,
        compiler_params=pltpu.CompilerParams(needs_layout_passes=False),
        scratch_types=[
            pltpu.VMEM((_ROWS_PER_W, _P), jnp.int32),
            pltpu.VMEM((_ROWS_PER_W // 2, 2 * _P), jnp.int32),
            pltpu.VMEM((_P,), jnp.int32),
            pltpu.VMEM((_NCHUNK,), jnp.int32),
            pltpu.VMEM((_L,), jnp.int32),
        ],
    )
    return f(v)


def kernel(x):
    del x
    v = pl.pallas_call(
        _vals_kernel,
        out_shape=jax.ShapeDtypeStruct((_R, _P), jnp.int32),
    )()
    return _sc_select(v).astype(jnp.bool_)


# speedup vs baseline: 1.0865x; 1.0090x over previous
import functools

import jax
import jax.numpy as jnp
from jax import lax
from jax.experimental import pallas as pl
from jax.experimental.pallas import tpu as pltpu
from jax.experimental.pallas import tpu_sc as plsc

_B = 128
_P = 1024
_R = 256

_KS0 = 0
_KS1 = 42
_KS2 = _KS0 ^ _KS1 ^ 0x1BD11BDA
_ROT = ((13, 15, 26, 6), (17, 29, 16, 24))

_NW = 32
_ROWS_PER_W = _R // _NW
_L = 16
_NCHUNK = _P // _L
_SENTINEL = 0x3FFFFFFF


def _rotl(x, r):
    return lax.shift_left(x, jnp.int32(r)) | lax.shift_right_logical(
        x, jnp.int32(32 - r)
    )


def _threefry_bits(x1):
    ks = (jnp.int32(_KS0), jnp.int32(_KS1), jnp.int32(_KS2))
    x0 = jnp.full(x1.shape, ks[0], jnp.int32)
    x1 = x1 + ks[1]
    for g in range(5):
        for r in _ROT[g % 2]:
            x0 = x0 + x1
            x1 = _rotl(x1, r)
            x1 = x1 ^ x0
        x0 = x0 + ks[(g + 1) % 3]
        x1 = x1 + ks[(g + 2) % 3] + jnp.int32(g + 1)
    return x0 ^ x1


def _vals_kernel(o_ref):
    half = _R // 2
    i = (
        (pl.program_id(0) * half + lax.broadcasted_iota(jnp.int32, (half, _P), 0))
        * _P
        + lax.broadcasted_iota(jnp.int32, (half, _P), 1)
    )
    o_ref[:] = lax.shift_right_logical(_threefry_bits(i), 9)


def _splat_to_scalar(x):
    return jnp.max(x)


def _select_body(v_hbm, out_hbm, v_rows, mask_rows, hist, hist64, candbuf):
    wid = lax.axis_index("s") * 2 + lax.axis_index("c")
    base = pl.multiple_of(wid * _ROWS_PER_W, _ROWS_PER_W)
    lane = lax.iota(jnp.int32, _L)
    zeros = jnp.zeros((_L,), jnp.int32)
    ones = jnp.ones((_L,), jnp.int32)

    pltpu.sync_copy(v_hbm.at[pl.ds(base, _ROWS_PER_W)], v_rows)

    def zero_body(c, _):
        hist[pl.ds(pl.multiple_of(c * _L, _L), _L)] = zeros
        return 0

    lax.fori_loop(0, _NCHUNK, zero_body, 0, unroll=2)
    for q in range(4):
        hist64[pl.ds(q * _L, _L)] = zeros

    def row_body(j, _):
        r = base + j
        k_row = jnp.where((r < _R // 2) == (j % 2 == 0), 768, 256)
        jh = lax.shift_right_logical(j, 1)
        half = (j & 1) * _P

        @plsc.parallel_loop(0, _NCHUNK, unroll=8)
        def _(c):
            vv = v_rows[j, pl.ds(pl.multiple_of(c * _L, _L), _L)]
            hi = lax.shift_right_logical(vv, 13)
            plsc.addupdate_scatter(hist, [hi], ones)
            plsc.addupdate_scatter(
                hist64, [lax.shift_right_logical(vv, 17)], ones
            )

        tot = jnp.int32(0)
        ch_vec = zeros
        lmax = zeros
        for q in range(4):
            s = plsc.cumsum(hist64[pl.ds(q * _L, _L)]) + tot
            ch_vec = ch_vec + plsc.all_reduce_population_count(s < k_row)
            lmax = jnp.maximum(lmax, jnp.where(s < k_row, s, 0))
            if q < 3:
                tot = jnp.max(s)
            hist64[pl.ds(q * _L, _L)] = zeros
        ch = jnp.max(ch_vec)
        l1 = jnp.max(lmax)

        s_fine = (
            plsc.cumsum(hist[pl.ds(pl.multiple_of(ch * _L, _L), _L)]) + l1
        )
        in_ch = s_fine < k_row
        g_local = plsc.all_reduce_population_count(in_ch)
        g = ch * _L + g_local
        nlt = jnp.max(jnp.where(lane < g_local, s_fine, l1))

        candbuf[:] = jnp.full((_L,), _SENTINEL, jnp.int32)

        @plsc.parallel_loop(0, _NCHUNK, unroll=8, carry=(zeros, lane))
        def _(c, carry):
            neq, pbase = carry
            off = pl.multiple_of(c * _L, _L)
            vv = v_rows[j, pl.ds(off, _L)]
            hist[pl.ds(off, _L)] = zeros
            hi = lax.shift_right_logical(vv, 13)
            m_lt = hi < g
            mask_rows[jh, pl.ds(half + off, _L)] = m_lt.astype(jnp.int32)
            m_eq = hi == g
            meq_i = m_eq.astype(jnp.int32)
            excl = plsc.cumsum(meq_i) - meq_i
            idx = jnp.minimum(excl + neq, _L - 1)
            comb = (vv & 0x1FFF) * _P + pbase
            plsc.store_scatter(candbuf, [idx], comb, mask=m_eq)
            neq = neq + plsc.all_reduce_population_count(m_eq)
            return neq, pbase + _L

        sk, _ = plsc.sort_key_val(candbuf[:], candbuf[:])
        selm = lane < (k_row - nlt)
        plsc.store_scatter(
            mask_rows,
            [jnp.full((_L,), 0, jnp.int32) + jh, half + (sk & (_P - 1))],
            ones,
            mask=selm,
        )
        return 0

    lax.fori_loop(0, _ROWS_PER_W, row_body, 0)

    pltpu.sync_copy(
        mask_rows,
        out_hbm.at[pl.ds(pl.multiple_of(wid * (_ROWS_PER_W // 2), _ROWS_PER_W // 2), _ROWS_PER_W // 2)],
    )


def _sc_select(v):
    mesh = plsc.VectorSubcoreMesh(core_axis_name="c", subcore_axis_name="s")
    f = pl.kernel(
        _select_body,
        out_type=jax.ShapeDtypeStruct((_B, 2 * _P), jnp.int32),
        mesh=mesh,
        compiler_params=pltpu.CompilerParams(needs_layout_passes=False),
        scratch_types=[
            pltpu.VMEM((_ROWS_PER_W, _P), jnp.int32),
            pltpu.VMEM((_ROWS_PER_W // 2, 2 * _P), jnp.int32),
            pltpu.VMEM((_P,), jnp.int32),
            pltpu.VMEM((_NCHUNK,), jnp.int32),
            pltpu.VMEM((_L,), jnp.int32),
        ],
    )
    return f(v)


def kernel(x):
    del x
    v = pl.pallas_call(
        _vals_kernel,
        grid=(2,),
        out_specs=pl.BlockSpec((_R // 2, _P), lambda i: (i, 0)),
        out_shape=jax.ShapeDtypeStruct((_R, _P), jnp.int32),
        compiler_params=pltpu.CompilerParams(
            dimension_semantics=("parallel",)
        ),
    )()
    return _sc_select(v).astype(jnp.bool_)
